# Initial kernel scaffold; baseline (speedup 1.0000x reference)
import jax
import jax.numpy as jnp
from jax.experimental import pallas as pl


def kernel(embeddings, W0, as0, ad0, b0, W1, as1, ad1, b1, W2, as2, ad2, b2, Wr, br):
    raise NotImplementedError("write your pallas kernel here")


# speedup vs baseline: 2388.1334x; 2388.1334x over previous
import jax
import jax.numpy as jnp
from jax.experimental import pallas as pl

_B, _N, _D = 4, 256, 64
_H, _HD = 4, 64


def _leaky_relu(x, slope=0.2):
    return jnp.where(x >= 0, x, slope * x)


def _gat_body(x_ref, W0, as0, ad0, b0, W1, as1, ad1, b1, W2, as2, ad2, b2,
              Wr, br, out_ref):
    N, H, C = _N, _H, _HD
    x = x_ref[0]
    for (W, a_s, a_d, b) in ((W0, as0, ad0, b0),
                             (W1, as1, ad1, b1),
                             (W2, as2, ad2, b2)):
        xp = jnp.dot(x, W[...], preferred_element_type=jnp.float32)
        acc = jnp.zeros((N, C), dtype=jnp.float32)
        for h in range(H):
            xph = xp[:, h * C:(h + 1) * C]
            a_d_h = a_d[h]
            a_s_h = a_s[h:h + 1, :]
            dh = (xph * a_d_h).sum(axis=-1, keepdims=True)
            sh = jax.lax.dot_general(
                a_s_h, xph,
                dimension_numbers=(((1,), (1,)), ((), ())),
                preferred_element_type=jnp.float32)
            logits = _leaky_relu(dh + sh)
            m = jnp.max(logits, axis=-1, keepdims=True)
            e = jnp.exp(logits - m)
            den = jnp.sum(e, axis=-1, keepdims=True)
            att = e / (den + 1e-16)
            acc = acc + jnp.dot(att, xph, preferred_element_type=jnp.float32)
        x = jax.nn.relu(acc * (1.0 / H) + b[...])
    pooled = jnp.mean(x, axis=0, keepdims=True)
    out_ref[0] = (jnp.dot(pooled, Wr[...], preferred_element_type=jnp.float32)
                  + br[...])


def kernel(embeddings, W0, as0, ad0, b0, W1, as1, ad1, b1, W2, as2, ad2, b2,
           Wr, br):
    full = lambda s: pl.BlockSpec(s, lambda b: tuple(0 for _ in s))
    out = pl.pallas_call(
        _gat_body,
        grid=(_B,),
        in_specs=[
            pl.BlockSpec((1, _N, _D), lambda b: (b, 0, 0)),
            full(W0.shape), full(as0.shape), full(ad0.shape), full(b0.shape),
            full(W1.shape), full(as1.shape), full(ad1.shape), full(b1.shape),
            full(W2.shape), full(as2.shape), full(ad2.shape), full(b2.shape),
            full(Wr.shape), full(br.shape),
        ],
        out_specs=pl.BlockSpec((1, 1, _D), lambda b: (b, 0, 0)),
        out_shape=jax.ShapeDtypeStruct((_B, 1, _D), jnp.float32),
    )(embeddings, W0, as0, ad0, b0, W1, as1, ad1, b1, W2, as2, ad2, b2,
      Wr, br)
    return out.reshape(_B, _D)
